# baseline (speedup 1.0000x reference)
import functools

import jax
import jax.numpy as jnp
from jax import lax
from jax.experimental import pallas as pl
from jax.experimental.pallas import tpu as pltpu
from jax.experimental.pallas import tpu_sc as plsc

C = 26
F = 13
V = 100000
D = 32

_INFO = plsc.get_sparse_core_info()
NC = _INFO.num_cores
NS = _INFO.num_subcores
NW = NC * NS
L16 = 16

IDXCH = 128


def _sc_body(T, TPW, cat_ref, cont_ref, tab_ref, w_ref, b_ref, out_ref,
             idx_v, rows_v, cval_v, crow_v, w_v, b_v, sem):
    nch = TPW // IDXCH
    wid = lax.axis_index("s") * NC + lax.axis_index("c")
    t0 = wid * TPW

    def field_body(f, carry):
        for j in range(nch):
            pltpu.sync_copy(cat_ref.at[f, pl.ds(t0 + j * IDXCH, IDXCH)],
                            idx_v.at[j])
        off = (f * V).astype(jnp.int32)
        for j in range(nch):
            for r in range(IDXCH // L16):
                sl = (j, pl.ds(r * L16, L16))
                idx_v[sl] = idx_v[sl] + off
        copies = [
            pltpu.async_copy(tab_ref.at[idx_v.at[j]],
                             rows_v.at[pl.ds(j * IDXCH, IDXCH)], sem)
            for j in range(nch)
        ]
        for cp in copies:
            cp.wait()
        pltpu.sync_copy(rows_v, out_ref.at[pl.ds(t0, TPW), f])
        return carry

    lax.fori_loop(0, C, field_body, 0)

    pltpu.sync_copy(w_ref, w_v)
    pltpu.sync_copy(b_ref, b_v)

    def cont_field(f, carry):
        pltpu.sync_copy(cont_ref.at[f, pl.ds(t0, TPW)], cval_v)
        w0 = w_v[f, pl.ds(0, L16)]
        w1 = w_v[f, pl.ds(L16, L16)]
        b0 = b_v[f, pl.ds(0, L16)]
        b1 = b_v[f, pl.ds(L16, L16)]

        def tok16(g, c2):
            t0g = g * L16
            vt = cval_v[pl.ds(t0g, L16)]
            for i in range(L16):
                v = vt[i]
                crow_v[t0g + i, pl.ds(0, L16)] = v * w0 + b0
                crow_v[t0g + i, pl.ds(L16, L16)] = v * w1 + b1
            return c2

        lax.fori_loop(0, TPW // L16, tok16, 0)
        pltpu.sync_copy(crow_v, out_ref.at[pl.ds(t0, TPW), C + f])
        return carry

    lax.fori_loop(0, F, cont_field, 0)


def kernel(cat, cont, tables, W, b):
    Bd, Ld, Cd = cat.shape
    T = Bd * Ld
    TPW = T // NW

    cat_t = cat.reshape(T, C).T
    cont_t = cont.reshape(T, F).T
    tab_f = tables.reshape(C * V, D)

    body = functools.partial(_sc_body, T, TPW)
    sc_call = pl.kernel(
        body,
        out_type=jax.ShapeDtypeStruct((T, C + F, D), jnp.float32),
        mesh=plsc.VectorSubcoreMesh(core_axis_name="c", subcore_axis_name="s"),
        scratch_types=[
            pltpu.VMEM((TPW // IDXCH, IDXCH), jnp.int32),
            pltpu.VMEM((TPW, D), jnp.float32),
            pltpu.VMEM((TPW,), jnp.float32),
            pltpu.VMEM((TPW, D), jnp.float32),
            pltpu.VMEM((F, D), jnp.float32),
            pltpu.VMEM((F, D), jnp.float32),
            pltpu.SemaphoreType.DMA,
        ],
        compiler_params=pltpu.CompilerParams(use_tc_tiling_on_sc=False),
        name="emb_layer_sc",
    )
    return sc_call(cat_t, cont_t, tab_f, W, b)


# speedup vs baseline: 4.5055x; 4.5055x over previous
import functools

import jax
import jax.numpy as jnp
from jax import lax
from jax.experimental import pallas as pl
from jax.experimental.pallas import tpu as pltpu
from jax.experimental.pallas import tpu_sc as plsc

C = 26
F = 13
V = 100000
D = 32

NC = 2
NS = 16
NW = NC * NS
L16 = 16
NQ = 4


def _sc_body(T, cat_ref, cont_ref, tab_ref, w_ref, b_ref, out_ref,
             row_v, q_v, qf_v, o_v, wrow_v, brow_v):
    TQ = T // NQ
    ng = TQ // L16
    wid = lax.axis_index("s") * NC + lax.axis_index("c")
    d = wid

    def cat_unit(c, carry):
        pltpu.sync_copy(tab_ref.at[c, d, :], row_v)
        for q in range(NQ):
            pltpu.sync_copy(cat_ref.at[c, pl.ds(q * TQ, TQ)], q_v)
            base = q * TQ

            @functools.partial(plsc.parallel_loop, 0, ng, unroll=8)
            def grp(g):
                iv = q_v[pl.ds(g * L16, L16)]
                o_v[pl.ds(base + g * L16, L16)] = plsc.load_gather(row_v, [iv])

        pltpu.sync_copy(o_v, out_ref.at[c, d, :])
        return carry

    lax.fori_loop(0, C, cat_unit, 0)

    dsplat = jnp.full((L16,), d, jnp.int32)

    def cont_unit(f, carry):
        pltpu.sync_copy(w_ref.at[f], wrow_v)
        pltpu.sync_copy(b_ref.at[f], brow_v)
        wv = plsc.load_gather(wrow_v, [dsplat])
        bv = plsc.load_gather(brow_v, [dsplat])
        for q in range(NQ):
            pltpu.sync_copy(cont_ref.at[f, pl.ds(q * TQ, TQ)], qf_v)
            base = q * TQ

            @functools.partial(plsc.parallel_loop, 0, ng, unroll=8)
            def grp(g):
                vv = qf_v[pl.ds(g * L16, L16)]
                o_v[pl.ds(base + g * L16, L16)] = vv * wv + bv

        pltpu.sync_copy(o_v, out_ref.at[C + f, d, :])
        return carry

    lax.fori_loop(0, F, cont_unit, 0)


def kernel(cat, cont, tables, W, b):
    Bd, Ld, Cd = cat.shape
    T = Bd * Ld

    tab_t = tables.transpose(0, 2, 1)
    cat_t = cat.reshape(T, C).T
    cont_t = cont.reshape(T, F).T

    body = functools.partial(_sc_body, T)
    sc_call = pl.kernel(
        body,
        out_type=jax.ShapeDtypeStruct((C + F, D, T), jnp.float32),
        mesh=plsc.VectorSubcoreMesh(core_axis_name="c", subcore_axis_name="s"),
        scratch_types=[
            pltpu.VMEM((V,), jnp.float32),
            pltpu.VMEM((T // NQ,), jnp.int32),
            pltpu.VMEM((T // NQ,), jnp.float32),
            pltpu.VMEM((T,), jnp.float32),
            pltpu.VMEM((D,), jnp.float32),
            pltpu.VMEM((D,), jnp.float32),
        ],
        name="emb_layer_sc",
    )
    out_t = sc_call(cat_t, cont_t, tab_t, W, b)
    return out_t.transpose(2, 0, 1)
